# baseline (speedup 1.0000x reference)
import dataclasses
import functools

import jax
import jax.numpy as jnp
from jax import lax
from jax.experimental import pallas as pl
from jax.experimental.pallas import tpu as pltpu
from jax.experimental.pallas import tpu_sc as plsc

B = 16384
F = 26
FACT = 16
L = 16
NC = 2
NS = 16
NW = NC * NS
RPW = B // NW
W = 64
NCHUNK = RPW // W


def kernel(x, emb_table, fc_table, lin_w, lin_b):
    xf = jnp.minimum(x, emb_table.shape[0] - 1).T.reshape(-1)
    fc_flat = fc_table.T.reshape(-1)
    w_vec = jnp.broadcast_to(lin_w.reshape(1), (L,)).astype(jnp.float32)
    b_vec = jnp.broadcast_to(lin_b.reshape(1), (L,)).astype(jnp.float32)

    mesh = plsc.VectorSubcoreMesh(core_axis_name="c", subcore_axis_name="s")
    cp = pltpu.CompilerParams()
    if "needs_layout_passes" in pltpu.CompilerParams.__dataclass_fields__:
        cp = dataclasses.replace(cp, needs_layout_passes=False)
    if "use_tc_tiling_on_sc" in pltpu.CompilerParams.__dataclass_fields__:
        cp = dataclasses.replace(cp, use_tc_tiling_on_sc=False)

    @functools.partial(
        pl.kernel,
        out_type=jax.ShapeDtypeStruct((B,), jnp.float32),
        mesh=mesh,
        compiler_params=cp,
        scratch_types=[
            pltpu.VMEM((W * F,), jnp.int32),
            pltpu.VMEM((W * F,), jnp.int32),
            pltpu.VMEM((W * F, FACT), jnp.float32),
            pltpu.VMEM((W * F, FACT), jnp.float32),
            pltpu.VMEM((W * F,), jnp.float32),
            pltpu.VMEM((W * F,), jnp.float32),
            pltpu.VMEM((RPW,), jnp.float32),
            pltpu.VMEM((L,), jnp.float32),
            pltpu.VMEM((L,), jnp.float32),
            pltpu.SemaphoreType.DMA,
            pltpu.SemaphoreType.DMA,
            pltpu.SemaphoreType.DMA,
            pltpu.SemaphoreType.DMA,
            pltpu.SemaphoreType.DMA,
            pltpu.SemaphoreType.DMA,
        ],
    )
    def sc_kernel(x_hbm, emb_hbm, fc_hbm, w_hbm, b_hbm, out_hbm,
                  idx0, idx1, rows0, rows1, fcv0, fcv1, sv, wv, bv,
                  si0, si1, se0, se1, sf0, sf1):
        wid = lax.axis_index("s") * NC + lax.axis_index("c")
        base = wid * RPW

        pltpu.sync_copy(w_hbm, wv)
        pltpu.sync_copy(b_hbm, bv)

        lanes = lax.iota(jnp.int32, L)
        wvec = wv[...]
        bvec = bv[...]

        bufs = ((idx0, rows0, fcv0, si0, se0, sf0),
                (idx1, rows1, fcv1, si1, se1, sf1))

        def idx_args(c, b):
            idx_v, _, _, si, _, _ = bufs[b]
            cb = base + c * W
            return [(x_hbm.at[pl.ds(f * B + cb, W)],
                     idx_v.at[pl.ds(f * W, W)], si) for f in range(F)]

        def issue_idx(c, b):
            for src, dst, sem in idx_args(c, b):
                pltpu.async_copy(src, dst, sem)

        def wait_idx(c, b):
            for src, dst, sem in idx_args(c, b):
                pltpu.make_async_copy(src, dst, sem).wait()

        def gather_args(b):
            idx_v, rows, fcv, _, se, sf = bufs[b]
            return ((emb_hbm.at[idx_v], rows, se),
                    (fc_hbm.at[idx_v], fcv, sf))

        def issue_gather(b):
            for src, dst, sem in gather_args(b):
                pltpu.async_copy(src, dst, sem)

        def wait_gather(b):
            for src, dst, sem in gather_args(b):
                pltpu.make_async_copy(src, dst, sem).wait()

        def compute(c, b):
            _, rows, fcv, _, _, _ = bufs[b]

            @pl.loop(0, W // L)
            def _group(g):
                fcs = fcv[pl.ds(g * L, L)]
                for f in range(1, F):
                    fcs = fcs + fcv[pl.ds(f * W + g * L, L)]

                zacc = jnp.zeros((L,), jnp.float32)
                for j in range(L):
                    rb = g * L + j
                    v = rows[rb, :]
                    acc = v
                    accsq = v * v
                    for f in range(1, F):
                        v = rows[f * W + rb, :]
                        acc = acc + v
                        accsq = accsq + v * v
                    inter = jnp.sum(acc * acc - accsq)
                    zacc = jnp.where(lanes == j, inter, zacc)

                z = 0.5 * zacc + wvec * fcs + bvec
                sv[pl.ds(c * W + g * L, L)] = 1.0 / (1.0 + jnp.exp(-z))

        issue_idx(0, 0)
        issue_idx(1, 1)
        wait_idx(0, 0)
        issue_gather(0)
        for c in range(NCHUNK):
            b = c % 2
            wait_gather(b)
            if c + 2 < NCHUNK:
                issue_idx(c + 2, b)
            if c + 1 < NCHUNK:
                wait_idx(c + 1, 1 - b)
                issue_gather(1 - b)
            compute(c, b)

        pltpu.sync_copy(sv, out_hbm.at[pl.ds(base, RPW)])

    out = sc_kernel(xf, emb_table, fc_flat, w_vec, b_vec)
    return out.reshape(B, 1)


# speedup vs baseline: 1.6320x; 1.6320x over previous
import dataclasses
import functools

import jax
import jax.numpy as jnp
from jax import lax
from jax.experimental import pallas as pl
from jax.experimental.pallas import tpu as pltpu
from jax.experimental.pallas import tpu_sc as plsc

B = 16384
F = 26
FACT = 16
L = 16
NC = 2
NS = 16
NW = NC * NS
RPW = B // NW
W = 16
NCHUNK = RPW // W

NN = 1000012
NPB = 2048
NBLK = -(-NN // NPB)
NPAD = NBLK * NPB
SROWS = NPAD // 8
BPT = -(-NBLK // NW)


def _cp(tc_tiling):
    cp = pltpu.CompilerParams()
    fields = pltpu.CompilerParams.__dataclass_fields__
    if "needs_layout_passes" in fields:
        cp = dataclasses.replace(cp, needs_layout_passes=False)
    if "use_tc_tiling_on_sc" in fields:
        cp = dataclasses.replace(cp, use_tc_tiling_on_sc=tc_tiling)
    if "disable_bounds_checks" in fields:
        cp = dataclasses.replace(cp, disable_bounds_checks=True)
    return cp


def kernel(x, emb_table, fc_table, lin_w, lin_b):
    xf = jnp.minimum(x, NN - 1).reshape(-1)
    fc_flat = fc_table.T.reshape(-1)
    embT = emb_table.T
    w_vec = jnp.broadcast_to(lin_w.reshape(1), (L,)).astype(jnp.float32)
    b_vec = jnp.broadcast_to(lin_b.reshape(1), (L,)).astype(jnp.float32)

    mesh = plsc.VectorSubcoreMesh(core_axis_name="c", subcore_axis_name="s")

    @functools.partial(
        pl.kernel,
        out_type=jax.ShapeDtypeStruct((NPAD * FACT,), jnp.float32),
        mesh=mesh,
        compiler_params=_cp(True),
        scratch_types=[
            pltpu.VMEM((FACT, NPB), jnp.float32),
            pltpu.VMEM((NPB * FACT,), jnp.float32),
            pltpu.SemaphoreType.DMA,
        ],
    )
    def t_kernel(emb_hbm, out_hbm, slab, tbuf, sem):
        wid = lax.axis_index("s") * NC + lax.axis_index("c")
        lanes = lax.iota(jnp.int32, L)

        for t in range(BPT):
            blk = wid * BPT + t

            @pl.when(blk < NBLK)
            def _():
                start = blk * NPB
                cp = pltpu.async_copy(
                    emb_hbm.at[pl.ds(0, FACT), pl.ds(start, NPB)], slab, sem)
                cp.wait()

                @pl.loop(0, NPB // L)
                def _t(g):
                    pos_base = (g * L + lanes) * FACT
                    for k in range(FACT):
                        v = slab[k, pl.ds(g * L, L)]
                        plsc.store_scatter(tbuf, [pos_base + k], v)

                pltpu.sync_copy(
                    tbuf, out_hbm.at[pl.ds(start * FACT, NPB * FACT)])

    embL = t_kernel(embT).reshape(SROWS, 128)

    @functools.partial(
        pl.kernel,
        out_type=jax.ShapeDtypeStruct((B,), jnp.float32),
        mesh=mesh,
        compiler_params=_cp(False),
        scratch_types=[
            pltpu.VMEM((W * F,), jnp.int32),
            pltpu.VMEM((W * F,), jnp.int32),
            pltpu.VMEM((W * F,), jnp.int32),
            pltpu.VMEM((W * F,), jnp.int32),
            pltpu.VMEM((W * F + L,), jnp.int32),
            pltpu.VMEM((W * F + L,), jnp.int32),
            pltpu.VMEM((W * F, 128), jnp.float32),
            pltpu.VMEM((W * F, 128), jnp.float32),
            pltpu.VMEM((W * F + L,), jnp.float32),
            pltpu.VMEM((W * F + L,), jnp.float32),
            pltpu.VMEM((RPW,), jnp.float32),
            pltpu.VMEM((L,), jnp.float32),
            pltpu.VMEM((L,), jnp.float32),
            pltpu.SemaphoreType.DMA,
            pltpu.SemaphoreType.DMA,
            pltpu.SemaphoreType.DMA,
            pltpu.SemaphoreType.DMA,
            pltpu.SemaphoreType.DMA,
            pltpu.SemaphoreType.DMA,
        ],
    )
    def g_kernel(x_hbm, emb_hbm, fc_hbm, w_hbm, b_hbm, out_hbm,
                 idx0, idx1, iq0, iq1, ic0, ic1, rows0, rows1, fcv0, fcv1,
                 sv, wv, bv, si0, si1, se0, se1, sf0, sf1):
        wid = lax.axis_index("s") * NC + lax.axis_index("c")
        base = wid * RPW

        pltpu.sync_copy(w_hbm, wv)
        pltpu.sync_copy(b_hbm, bv)

        lanes = lax.iota(jnp.int32, L)
        tail_mask = lanes < (F - L)
        wvec = wv[...]
        bvec = bv[...]

        bufs = ((idx0, iq0, ic0, rows0, fcv0, si0, se0, sf0),
                (idx1, iq1, ic1, rows1, fcv1, si1, se1, sf1))

        def idx_args(c, b):
            idx_v = bufs[b][0]
            si = bufs[b][5]
            cb = (base + c * W) * F
            return [(x_hbm.at[pl.ds(cb, W * F)], idx_v, si)]

        def issue_idx(c, b):
            for src, dst, sem in idx_args(c, b):
                pltpu.async_copy(src, dst, sem)

        def wait_idx(c, b):
            for src, dst, sem in idx_args(c, b):
                pltpu.make_async_copy(src, dst, sem).wait()

        def prep_idx(b):
            idx_v, iq, ic = bufs[b][0], bufs[b][1], bufs[b][2]

            @pl.loop(0, W * F, step=L)
            def _p(i):
                v = idx_v[pl.ds(i, L)]
                iq[pl.ds(i, L)] = v >> 3
                ic[pl.ds(i, L)] = (v & 7) * FACT

        def gather_args(b):
            idx_v, iq, _, rows, fcv = bufs[b][:5]
            se, sf = bufs[b][6], bufs[b][7]
            return ((emb_hbm.at[iq], rows, se),
                    (fc_hbm.at[idx_v], fcv.at[pl.ds(0, W * F)], sf))

        def issue_gather(b):
            for src, dst, sem in gather_args(b):
                pltpu.async_copy(src, dst, sem)

        def wait_gather(b):
            for src, dst, sem in gather_args(b):
                pltpu.make_async_copy(src, dst, sem).wait()

        def compute(c, b):
            _, _, ic, rows, fcv = bufs[b][:5]

            zacc = jnp.zeros((L,), jnp.float32)
            facc = jnp.zeros((L,), jnp.float32)
            for j in range(L):
                rb = j * F
                cols_lo = ic[pl.ds(rb, L)]
                cols_hi = ic[pl.ds(rb + L, L)]
                acc = None
                accsq = None
                for f in range(F):
                    col = cols_lo[f] if f < L else cols_hi[f - L]
                    v = rows[rb + f, pl.ds(col, L)]
                    if acc is None:
                        acc = v
                        accsq = v * v
                    else:
                        acc = acc + v
                        accsq = accsq + v * v
                inter = jnp.sum(acc * acc - accsq)

                f1 = fcv[pl.ds(rb, L)]
                f2 = fcv[pl.ds(rb + L, L)]
                f2 = jnp.where(tail_mask, f2, 0.0)
                fcs = jnp.sum(f1 + f2)

                zacc = jnp.where(lanes == j, inter, zacc)
                facc = jnp.where(lanes == j, fcs, facc)

            z = 0.5 * zacc + wvec * facc + bvec
            sv[pl.ds(c * W, L)] = 1.0 / (1.0 + jnp.exp(-z))

        issue_idx(0, 0)
        issue_idx(1, 1)
        wait_idx(0, 0)
        prep_idx(0)
        issue_gather(0)

        @pl.loop(0, NCHUNK, step=2)
        def _chunks(c):
            for step in range(2):
                cc = c + step
                b = step
                wait_gather(b)

                @pl.when(cc + 2 < NCHUNK)
                def _():
                    issue_idx(cc + 2, b)

                @pl.when(cc + 1 < NCHUNK)
                def _():
                    wait_idx(cc + 1, 1 - b)
                    prep_idx(1 - b)
                    issue_gather(1 - b)

                compute(cc, b)

        pltpu.sync_copy(sv, out_hbm.at[pl.ds(base, RPW)])

    out = g_kernel(xf, embL, fc_flat, w_vec, b_vec)
    return out.reshape(B, 1)
